# Initial kernel scaffold; baseline (speedup 1.0000x reference)
import jax
import jax.numpy as jnp
from jax.experimental import pallas as pl


def kernel(label):
    raise NotImplementedError("write your pallas kernel here")


# speedup vs baseline: 1.0316x; 1.0316x over previous
import functools

import jax
import jax.numpy as jnp
from jax import lax
from jax.experimental import pallas as pl
from jax.experimental.pallas import tpu as pltpu
from jax.experimental.pallas import tpu_sc as plsc

N_CLASSES = 1000
LB_SMOOTH = 0.1
LB_POS = 1.0 - LB_SMOOTH
LB_NEG = LB_SMOOTH / N_CLASSES

B = 16384
NW = 32
ROWS_PER_W = B // NW
CH = 16
CHUNKS = ROWS_PER_W // CH
CHUNK_ELEMS = CH * N_CLASSES


def _onehot_body(label_hbm, out_hbm, lbl_v, buf0, buf1, sem0, sem1):
    wid = lax.axis_index("s") * 2 + lax.axis_index("c")
    base_row = wid * ROWS_PER_W

    pltpu.sync_copy(label_hbm.at[pl.ds(base_row, ROWS_PER_W)], lbl_v)

    neg = jnp.full((16,), LB_NEG, dtype=jnp.float32)
    pos = jnp.full((16,), LB_POS, dtype=jnp.float32)
    row_off = lax.iota(jnp.int32, 16) * N_CLASSES

    def fill(j, _):
        buf0[pl.ds(j * 16, 16)] = neg
        buf1[pl.ds(j * 16, 16)] = neg
        return 0
    lax.fori_loop(0, CHUNK_ELEMS // 16, fill, 0)

    bufs = (buf0, buf1)
    sems = (sem0, sem1)
    copies = [None, None]

    for c in range(CHUNKS):
        b = c & 1
        buf = bufs[b]
        if copies[b] is not None:
            copies[b].wait()
            prev_lbl = lbl_v[pl.ds((c - 2) * CH, 16)]
            plsc.store_scatter(buf, [row_off + prev_lbl], neg)
        lbl = lbl_v[pl.ds(c * CH, 16)]
        plsc.store_scatter(buf, [row_off + lbl], pos)
        copies[b] = pltpu.async_copy(
            buf,
            out_hbm.at[pl.ds((base_row + c * CH) * N_CLASSES, CHUNK_ELEMS)],
            sems[b],
        )
    copies[0].wait()
    copies[1].wait()


@functools.partial(jax.jit, static_argnames=())
def kernel(label):
    mesh = plsc.VectorSubcoreMesh(core_axis_name="c", subcore_axis_name="s")
    flat = pl.kernel(
        _onehot_body,
        out_type=jax.ShapeDtypeStruct((B * N_CLASSES,), jnp.float32),
        mesh=mesh,
        scratch_types=[
            pltpu.VMEM((ROWS_PER_W,), jnp.int32),
            pltpu.VMEM((CHUNK_ELEMS,), jnp.float32),
            pltpu.VMEM((CHUNK_ELEMS,), jnp.float32),
            pltpu.SemaphoreType.DMA,
            pltpu.SemaphoreType.DMA,
        ],
        compiler_params=pltpu.CompilerParams(needs_layout_passes=False),
    )(label)
    return flat.reshape(B, N_CLASSES)


# speedup vs baseline: 1.6274x; 1.5775x over previous
import jax
import jax.numpy as jnp
from jax import lax
from jax.experimental import pallas as pl
from jax.experimental.pallas import tpu as pltpu
from jax.experimental.pallas import tpu_sc as plsc

N_CLASSES = 1000
C_PAD = 1024
LB_SMOOTH = 0.1
LB_POS = 1.0 - LB_SMOOTH
LB_NEG = LB_SMOOTH / N_CLASSES

B = 16384
NW = 32
ROWS_PER_W = B // NW
CH = 16
CHUNKS = ROWS_PER_W // CH


def _onehot_body(label_hbm, out_hbm, lbl_v, buf0, buf1, sem0, sem1):
    wid = lax.axis_index("s") * 2 + lax.axis_index("c")
    base_row = wid * ROWS_PER_W

    pltpu.sync_copy(label_hbm.at[pl.ds(base_row, ROWS_PER_W)], lbl_v)

    neg = jnp.full((16,), LB_NEG, dtype=jnp.float32)
    pos = jnp.full((16,), LB_POS, dtype=jnp.float32)
    row_iota = lax.iota(jnp.int32, 16)

    def fill(k, _):
        row = k // 63
        off = jnp.minimum((k % 63) * 16, N_CLASSES - 16)
        buf0[row, pl.ds(off, 16)] = neg
        buf1[row, pl.ds(off, 16)] = neg
        return 0
    lax.fori_loop(0, CH * 63, fill, 0)

    bufs = (buf0, buf1)
    sems = (sem0, sem1)
    copies = [None, None]

    for c in range(CHUNKS):
        b = c & 1
        buf = bufs[b]
        if copies[b] is not None:
            copies[b].wait()
            prev_lbl = lbl_v[pl.ds((c - 2) * CH, 16)]
            plsc.store_scatter(buf, [row_iota, prev_lbl], neg)
        lbl = lbl_v[pl.ds(c * CH, 16)]
        plsc.store_scatter(buf, [row_iota, lbl], pos)
        copies[b] = pltpu.async_copy(
            buf,
            out_hbm.at[pl.ds(base_row + c * CH, CH), :],
            sems[b],
        )
    copies[0].wait()
    copies[1].wait()


@jax.jit
def kernel(label):
    mesh = plsc.VectorSubcoreMesh(core_axis_name="c", subcore_axis_name="s")
    return pl.kernel(
        _onehot_body,
        out_type=jax.ShapeDtypeStruct((B, N_CLASSES), jnp.float32),
        mesh=mesh,
        scratch_types=[
            pltpu.VMEM((ROWS_PER_W,), jnp.int32),
            pltpu.VMEM((CH, N_CLASSES), jnp.float32),
            pltpu.VMEM((CH, N_CLASSES), jnp.float32),
            pltpu.SemaphoreType.DMA,
            pltpu.SemaphoreType.DMA,
        ],
        compiler_params=pltpu.CompilerParams(needs_layout_passes=False),
    )(label)
